# Initial kernel scaffold; baseline (speedup 1.0000x reference)
import jax
import jax.numpy as jnp
from jax.experimental import pallas as pl


def kernel(edge_index, emb_weight):
    raise NotImplementedError("write your pallas kernel here")


# speedup vs baseline: 3.4769x; 3.4769x over previous
import jax
import jax.numpy as jnp
from jax import lax
from jax.experimental import pallas as pl
from jax.experimental.pallas import tpu as pltpu
from jax.experimental.pallas import tpu_sc as plsc

N_NODES = 50000
EMB_DIM = 64
QW = 16
N_LAYERS = 3
N_EDGES = 800000

NC = 2
NS = 16
LANES = 16

NODE_PAD = 51200
SLICE = NODE_PAD // NS
EDGE_PAD = 819200
BLOCKS = EDGE_PAD // 128
TILE_BLOCKS = BLOCKS // NS
CHUNKS = TILE_BLOCKS // 8
DUMMY = N_NODES

_mesh = plsc.VectorSubcoreMesh(
    core_axis_name="c", subcore_axis_name="s", num_cores=NC, num_subcores=NS
)

_f32 = jnp.float32


def _spmm_body(u_hbm, row_hbm, col_hbm, out_hbm, rowbuf, colbuf, rows, zbuf, acc, sem):
    c = lax.axis_index("c")
    s = lax.axis_index("s")
    z16 = jnp.zeros((LANES,), _f32)

    @pl.loop(0, SLICE // 4, unroll=4)
    def _(i):
        zbuf[i, pl.ds(0, LANES)] = z16

    base = s * SLICE
    for j in range(4):
        pltpu.sync_copy(zbuf, acc.at[pl.ds(base + j * (SLICE // 4), SLICE // 4)])
    plsc.subcore_barrier()

    tb = s * TILE_BLOCKS

    @pl.loop(0, CHUNKS)
    def _(i):
        bb = tb + i * 8
        pltpu.sync_copy(row_hbm.at[pl.ds(bb, 8)], rowbuf)
        pltpu.sync_copy(col_hbm.at[pl.ds(bb, 8)], colbuf)
        handles = [
            pltpu.async_copy(
                u_hbm.at[c].at[rowbuf.at[j]], rows.at[pl.ds(j * 128, 128)], sem
            )
            for j in range(8)
        ]
        for h in handles:
            h.wait()
        for j in range(8):
            pltpu.sync_copy(
                rows.at[pl.ds(j * 128, 128)], acc.at[colbuf.at[j]], add=True
            )

    plsc.subcore_barrier()
    pltpu.sync_copy(acc.at[pl.ds(base, SLICE)], out_hbm.at[c].at[pl.ds(base, SLICE)])


_spmm_kernel = pl.kernel(
    _spmm_body,
    out_type=jax.ShapeDtypeStruct((NC, NODE_PAD, QW), _f32),
    mesh=_mesh,
    compiler_params=pltpu.CompilerParams(use_tc_tiling_on_sc=False),
    scratch_types=[
        pltpu.VMEM((8, 128), jnp.int32),
        pltpu.VMEM((8, 128), jnp.int32),
        pltpu.VMEM((1024, QW), _f32),
        pltpu.VMEM((SLICE // 4, QW), _f32),
        pltpu.VMEM_SHARED((NODE_PAD, QW), _f32),
        pltpu.SemaphoreType.DMA,
    ],
)


_BN = 1600
_TC_GRID = NODE_PAD // _BN

_qspec = pl.BlockSpec((NC, _BN, QW), lambda i: (0, i, 0))
_q_shape = jax.ShapeDtypeStruct((NC, NODE_PAD, QW), _f32)


def _prep_body(deg_ref, emb_ref, d_ref, dsq_ref, ua_ref, ub_ref):
    g = deg_ref[...][0, :, :1]
    d = jnp.where(g > 0.0, lax.rsqrt(g), 0.0)
    d_ref[...] = d
    dsq_ref[...] = d * d
    u = d * emb_ref[...]
    ua_ref[0] = u[:, 0 * QW : 1 * QW]
    ua_ref[1] = u[:, 1 * QW : 2 * QW]
    ub_ref[0] = u[:, 2 * QW : 3 * QW]
    ub_ref[1] = u[:, 3 * QW : 4 * QW]


def _prep_call(deg, emb_p):
    return pl.pallas_call(
        _prep_body,
        grid=(_TC_GRID,),
        in_specs=[
            _qspec,
            pl.BlockSpec((_BN, EMB_DIM), lambda i: (i, 0)),
        ],
        out_specs=[
            pl.BlockSpec((_BN, 1), lambda i: (i, 0)),
            pl.BlockSpec((_BN, 1), lambda i: (i, 0)),
            _qspec,
            _qspec,
        ],
        out_shape=[
            jax.ShapeDtypeStruct((NODE_PAD, 1), _f32),
            jax.ShapeDtypeStruct((NODE_PAD, 1), _f32),
            _q_shape,
            _q_shape,
        ],
    )(deg, emb_p)


def _scale_body(ta_ref, tb_ref, dsq_ref, ua_ref, ub_ref):
    dsq = dsq_ref[...][None]
    ua_ref[...] = dsq * ta_ref[...]
    ub_ref[...] = dsq * tb_ref[...]


def _scale_call(ta, tb, dsq2):
    return pl.pallas_call(
        _scale_body,
        grid=(_TC_GRID,),
        in_specs=[_qspec, _qspec, pl.BlockSpec((_BN, 1), lambda i: (i, 0))],
        out_specs=[_qspec, _qspec],
        out_shape=[_q_shape, _q_shape],
    )(ta, tb, dsq2)


def _final_body(emb_ref, d_ref, t1a, t1b, t2a, t2b, t3a, t3b, out_ref):
    sa = t1a[...] + t2a[...] + t3a[...]
    sb = t1b[...] + t2b[...] + t3b[...]
    cat = jnp.concatenate([sa[0], sa[1], sb[0], sb[1]], axis=-1)
    out_ref[...] = 0.25 * (emb_ref[...] + d_ref[...] * cat)


def _final_call(emb_p, d2, ts):
    return pl.pallas_call(
        _final_body,
        grid=(_TC_GRID,),
        in_specs=[
            pl.BlockSpec((_BN, EMB_DIM), lambda i: (i, 0)),
            pl.BlockSpec((_BN, 1), lambda i: (i, 0)),
        ]
        + [_qspec] * 6,
        out_specs=pl.BlockSpec((_BN, EMB_DIM), lambda i: (i, 0)),
        out_shape=jax.ShapeDtypeStruct((NODE_PAD, EMB_DIM), _f32),
    )(emb_p, d2, *ts)


@jax.jit
def kernel(edge_index, emb_weight):
    row = edge_index[0]
    col = edge_index[1]
    pad = EDGE_PAD - N_EDGES
    row2d = jnp.concatenate([row, jnp.zeros((pad,), jnp.int32)]).reshape(BLOCKS, 128)
    col2d = jnp.concatenate([col, jnp.full((pad,), DUMMY, jnp.int32)]).reshape(
        BLOCKS, 128
    )
    emb_p = jnp.zeros((NODE_PAD, EMB_DIM), _f32).at[:N_NODES].set(emb_weight)

    ones_u = jnp.ones((NC, NODE_PAD, QW), _f32)
    zrow = jnp.zeros((BLOCKS, 128), jnp.int32)
    deg = _spmm_kernel(ones_u, zrow, col2d)
    d2, dsq2, ua, ub = _prep_call(deg, emb_p)

    ts = []
    for layer in range(N_LAYERS):
        ta = _spmm_kernel(ua, row2d, col2d)
        tb = _spmm_kernel(ub, row2d, col2d)
        ts += [ta, tb]
        if layer < N_LAYERS - 1:
            ua, ub = _scale_call(ta, tb, dsq2)

    out = _final_call(emb_p, d2, ts)
    return out[:N_NODES]


# speedup vs baseline: 3.8597x; 1.1101x over previous
import jax
import jax.numpy as jnp
from jax import lax
from jax.experimental import pallas as pl
from jax.experimental.pallas import tpu as pltpu
from jax.experimental.pallas import tpu_sc as plsc

N_NODES = 50000
EMB_DIM = 64
QW = 16
N_LAYERS = 3
N_EDGES = 800000

NC = 2
NS = 16
LANES = 16

NODE_PAD = 51200
SLICE = NODE_PAD // NS
EDGE_PAD = 819200
BLOCKS = EDGE_PAD // 128
TILE_BLOCKS = BLOCKS // NS
CHUNKS = TILE_BLOCKS // 8
DUMMY = N_NODES

_mesh = plsc.VectorSubcoreMesh(
    core_axis_name="c", subcore_axis_name="s", num_cores=NC, num_subcores=NS
)

_f32 = jnp.float32


CB = 10
NCH = TILE_BLOCKS // CB
CE = CB * 128


def _spmm_body(
    u_hbm, row_hbm, col_hbm, out_hbm,
    ridx0, cidx0, ridx1, cidx1, rows0, rows1, zbuf, acc,
    semg0, semg1, sems0, sems1, semi0,
):
    c = lax.axis_index("c")
    s = lax.axis_index("s")
    z16 = jnp.zeros((LANES,), _f32)

    @pl.loop(0, SLICE // 4, unroll=4)
    def _(i):
        zbuf[i, pl.ds(0, LANES)] = z16

    base = s * SLICE
    for j in range(4):
        pltpu.sync_copy(zbuf, acc.at[pl.ds(base + j * (SLICE // 4), SLICE // 4)])
    plsc.subcore_barrier()

    tb = s * TILE_BLOCKS

    def load_idx_sync(ci, ridx, cidx):
        bb = tb + ci * CB
        pltpu.sync_copy(row_hbm.at[pl.ds(bb, CB)], ridx)
        pltpu.sync_copy(col_hbm.at[pl.ds(bb, CB)], cidx)

    def fire_gathers(ridx, rows, semg):
        for j in range(CB):
            pltpu.async_copy(
                u_hbm.at[c].at[ridx.at[j]], rows.at[pl.ds(j * 128, 128)], semg
            )

    def wait_gathers(rows, semg):
        pltpu.make_async_copy(u_hbm.at[c].at[pl.ds(0, CE)], rows, semg).wait()

    def fire_scatters(cidx, rows, sems):
        for j in range(CB):
            pltpu.async_copy(
                rows.at[pl.ds(j * 128, 128)], acc.at[cidx.at[j]], sems, add=True
            )

    def wait_scatters(rows, sems):
        pltpu.make_async_copy(rows, acc.at[pl.ds(0, CE)], sems).wait()

    load_idx_sync(0, ridx0, cidx0)
    fire_gathers(ridx0, rows0, semg0)

    @pl.loop(0, NCH // 2)
    def _(g):
        @pl.when(g > 0)
        def _():
            wait_scatters(rows1, sems1)

        load_idx_sync(2 * g + 1, ridx1, cidx1)
        fire_gathers(ridx1, rows1, semg1)

        wait_gathers(rows0, semg0)
        fire_scatters(cidx0, rows0, sems0)

        @pl.when(g < NCH // 2 - 1)
        def _():
            wait_scatters(rows0, sems0)
            load_idx_sync(2 * g + 2, ridx0, cidx0)
            fire_gathers(ridx0, rows0, semg0)

        wait_gathers(rows1, semg1)
        fire_scatters(cidx1, rows1, sems1)

    wait_scatters(rows0, sems0)
    wait_scatters(rows1, sems1)
    plsc.subcore_barrier()
    pltpu.sync_copy(acc.at[pl.ds(base, SLICE)], out_hbm.at[c].at[pl.ds(base, SLICE)])


_spmm_kernel = pl.kernel(
    _spmm_body,
    out_type=jax.ShapeDtypeStruct((NC, NODE_PAD, QW), _f32),
    mesh=_mesh,
    compiler_params=pltpu.CompilerParams(use_tc_tiling_on_sc=False),
    scratch_types=[
        pltpu.VMEM((CB, 128), jnp.int32),
        pltpu.VMEM((CB, 128), jnp.int32),
        pltpu.VMEM((CB, 128), jnp.int32),
        pltpu.VMEM((CB, 128), jnp.int32),
        pltpu.VMEM((CE, QW), _f32),
        pltpu.VMEM((CE, QW), _f32),
        pltpu.VMEM((SLICE // 4, QW), _f32),
        pltpu.VMEM_SHARED((NODE_PAD, QW), _f32),
        pltpu.SemaphoreType.DMA,
        pltpu.SemaphoreType.DMA,
        pltpu.SemaphoreType.DMA,
        pltpu.SemaphoreType.DMA,
        pltpu.SemaphoreType.DMA,
    ],
)


_BN = 1600
_TC_GRID = NODE_PAD // _BN

_qspec = pl.BlockSpec((NC, _BN, QW), lambda i: (0, i, 0))
_q_shape = jax.ShapeDtypeStruct((NC, NODE_PAD, QW), _f32)


def _prep_body(deg_ref, emb_ref, d_ref, dsq_ref, ua_ref, ub_ref):
    g = deg_ref[...][0, :, :1]
    d = jnp.where(g > 0.0, lax.rsqrt(g), 0.0)
    d_ref[...] = d
    dsq_ref[...] = d * d
    u = d * emb_ref[...]
    ua_ref[0] = u[:, 0 * QW : 1 * QW]
    ua_ref[1] = u[:, 1 * QW : 2 * QW]
    ub_ref[0] = u[:, 2 * QW : 3 * QW]
    ub_ref[1] = u[:, 3 * QW : 4 * QW]


def _prep_call(deg, emb_p):
    return pl.pallas_call(
        _prep_body,
        grid=(_TC_GRID,),
        in_specs=[
            _qspec,
            pl.BlockSpec((_BN, EMB_DIM), lambda i: (i, 0)),
        ],
        out_specs=[
            pl.BlockSpec((_BN, 1), lambda i: (i, 0)),
            pl.BlockSpec((_BN, 1), lambda i: (i, 0)),
            _qspec,
            _qspec,
        ],
        out_shape=[
            jax.ShapeDtypeStruct((NODE_PAD, 1), _f32),
            jax.ShapeDtypeStruct((NODE_PAD, 1), _f32),
            _q_shape,
            _q_shape,
        ],
    )(deg, emb_p)


def _scale_body(ta_ref, tb_ref, dsq_ref, ua_ref, ub_ref):
    dsq = dsq_ref[...][None]
    ua_ref[...] = dsq * ta_ref[...]
    ub_ref[...] = dsq * tb_ref[...]


def _scale_call(ta, tb, dsq2):
    return pl.pallas_call(
        _scale_body,
        grid=(_TC_GRID,),
        in_specs=[_qspec, _qspec, pl.BlockSpec((_BN, 1), lambda i: (i, 0))],
        out_specs=[_qspec, _qspec],
        out_shape=[_q_shape, _q_shape],
    )(ta, tb, dsq2)


def _final_body(emb_ref, d_ref, t1a, t1b, t2a, t2b, t3a, t3b, out_ref):
    sa = t1a[...] + t2a[...] + t3a[...]
    sb = t1b[...] + t2b[...] + t3b[...]
    cat = jnp.concatenate([sa[0], sa[1], sb[0], sb[1]], axis=-1)
    out_ref[...] = 0.25 * (emb_ref[...] + d_ref[...] * cat)


def _final_call(emb_p, d2, ts):
    return pl.pallas_call(
        _final_body,
        grid=(_TC_GRID,),
        in_specs=[
            pl.BlockSpec((_BN, EMB_DIM), lambda i: (i, 0)),
            pl.BlockSpec((_BN, 1), lambda i: (i, 0)),
        ]
        + [_qspec] * 6,
        out_specs=pl.BlockSpec((_BN, EMB_DIM), lambda i: (i, 0)),
        out_shape=jax.ShapeDtypeStruct((NODE_PAD, EMB_DIM), _f32),
    )(emb_p, d2, *ts)


@jax.jit
def kernel(edge_index, emb_weight):
    row = edge_index[0]
    col = edge_index[1]
    pad = EDGE_PAD - N_EDGES
    row2d = jnp.concatenate([row, jnp.zeros((pad,), jnp.int32)]).reshape(BLOCKS, 128)
    col2d = jnp.concatenate([col, jnp.full((pad,), DUMMY, jnp.int32)]).reshape(
        BLOCKS, 128
    )
    emb_p = jnp.zeros((NODE_PAD, EMB_DIM), _f32).at[:N_NODES].set(emb_weight)

    ones_u = jnp.ones((NC, NODE_PAD, QW), _f32)
    zrow = jnp.zeros((BLOCKS, 128), jnp.int32)
    deg = _spmm_kernel(ones_u, zrow, col2d)
    d2, dsq2, ua, ub = _prep_call(deg, emb_p)

    ts = []
    for layer in range(N_LAYERS):
        ta = _spmm_kernel(ua, row2d, col2d)
        tb = _spmm_kernel(ub, row2d, col2d)
        ts += [ta, tb]
        if layer < N_LAYERS - 1:
            ua, ub = _scale_call(ta, tb, dsq2)

    out = _final_call(emb_p, d2, ts)
    return out[:N_NODES]


# speedup vs baseline: 10.4907x; 2.7180x over previous
import jax
import jax.numpy as jnp
from jax import lax
from jax.experimental import pallas as pl
from jax.experimental.pallas import tpu as pltpu
from jax.experimental.pallas import tpu_sc as plsc

N_NODES = 50000
EMB_DIM = 64
QW = 16
N_LAYERS = 3
N_EDGES = 800000

NC = 2
NS = 16
LANES = 16

NODE_PAD = 51200
SLICE = NODE_PAD // NS
EDGE_PAD = 819200
BLOCKS = EDGE_PAD // 128
TILE_BLOCKS = BLOCKS // NS
CHUNKS = TILE_BLOCKS // 8
DUMMY = N_NODES

_mesh = plsc.VectorSubcoreMesh(
    core_axis_name="c", subcore_axis_name="s", num_cores=NC, num_subcores=NS
)

_f32 = jnp.float32


CB = 10
NCH = TILE_BLOCKS // CB
CE = CB * 128


def _spmm_body(
    u_hbm, row_hbm, col_hbm, out_hbm,
    ridx0, cidx0, ridx1, cidx1, rows0, rows1, zbuf, acc,
    semg0, semg1, sems0, sems1, semi0,
):
    c = lax.axis_index("c")
    s = lax.axis_index("s")
    z16 = jnp.zeros((LANES,), _f32)

    @pl.loop(0, SLICE // 4, unroll=4)
    def _(i):
        zbuf[i, pl.ds(0, LANES)] = z16

    base = s * SLICE
    for j in range(4):
        pltpu.sync_copy(zbuf, acc.at[pl.ds(base + j * (SLICE // 4), SLICE // 4)])
    plsc.subcore_barrier()

    tb = s * TILE_BLOCKS

    def load_idx_sync(ci, ridx, cidx):
        bb = tb + ci * CB
        pltpu.sync_copy(row_hbm.at[pl.ds(bb, CB)], ridx)
        pltpu.sync_copy(col_hbm.at[pl.ds(bb, CB)], cidx)

    def fire_gathers(ridx, rows, semg):
        for j in range(CB):
            pltpu.async_copy(
                u_hbm.at[c].at[ridx.at[j]], rows.at[pl.ds(j * 128, 128)], semg
            )

    def wait_gathers(rows, semg):
        pltpu.make_async_copy(u_hbm.at[c].at[pl.ds(0, CE)], rows, semg).wait()

    def fire_scatters(cidx, rows, sems):
        for j in range(CB):
            pltpu.async_copy(
                rows.at[pl.ds(j * 128, 128)], acc.at[cidx.at[j]], sems, add=True
            )

    def wait_scatters(rows, sems):
        pltpu.make_async_copy(rows, acc.at[pl.ds(0, CE)], sems).wait()

    load_idx_sync(0, ridx0, cidx0)
    fire_gathers(ridx0, rows0, semg0)

    @pl.loop(0, NCH // 2)
    def _(g):
        @pl.when(g > 0)
        def _():
            wait_scatters(rows1, sems1)

        load_idx_sync(2 * g + 1, ridx1, cidx1)
        fire_gathers(ridx1, rows1, semg1)

        wait_gathers(rows0, semg0)
        fire_scatters(cidx0, rows0, sems0)

        @pl.when(g < NCH // 2 - 1)
        def _():
            wait_scatters(rows0, sems0)
            load_idx_sync(2 * g + 2, ridx0, cidx0)
            fire_gathers(ridx0, rows0, semg0)

        wait_gathers(rows1, semg1)
        fire_scatters(cidx1, rows1, sems1)

    wait_scatters(rows0, sems0)
    wait_scatters(rows1, sems1)
    plsc.subcore_barrier()
    pltpu.sync_copy(acc.at[pl.ds(base, SLICE)], out_hbm.at[c].at[pl.ds(base, SLICE)])


_spmm_kernel = pl.kernel(
    _spmm_body,
    out_type=jax.ShapeDtypeStruct((NC, NODE_PAD, QW), _f32),
    mesh=_mesh,
    compiler_params=pltpu.CompilerParams(use_tc_tiling_on_sc=False),
    scratch_types=[
        pltpu.VMEM((CB, 128), jnp.int32),
        pltpu.VMEM((CB, 128), jnp.int32),
        pltpu.VMEM((CB, 128), jnp.int32),
        pltpu.VMEM((CB, 128), jnp.int32),
        pltpu.VMEM((CE, QW), _f32),
        pltpu.VMEM((CE, QW), _f32),
        pltpu.VMEM((SLICE // 4, QW), _f32),
        pltpu.VMEM_SHARED((NODE_PAD, QW), _f32),
        pltpu.SemaphoreType.DMA,
        pltpu.SemaphoreType.DMA,
        pltpu.SemaphoreType.DMA,
        pltpu.SemaphoreType.DMA,
        pltpu.SemaphoreType.DMA,
    ],
)


_BN = 1600
_TC_GRID = NODE_PAD // _BN

_qspec = pl.BlockSpec((NC, _BN, QW), lambda i: (0, i, 0))
_q_shape = jax.ShapeDtypeStruct((NC, NODE_PAD, QW), _f32)


def _prep_body(deg_ref, emb_ref, d_ref, dsq_ref, ua_ref, ub_ref):
    g = deg_ref[...][0, :, :1]
    d = jnp.where(g > 0.0, lax.rsqrt(g), 0.0)
    d_ref[...] = d
    dsq_ref[...] = d * d
    u = d * emb_ref[...]
    ua_ref[0] = u[:, 0 * QW : 1 * QW]
    ua_ref[1] = u[:, 1 * QW : 2 * QW]
    ub_ref[0] = u[:, 2 * QW : 3 * QW]
    ub_ref[1] = u[:, 3 * QW : 4 * QW]


def _prep_call(deg, emb_p):
    return pl.pallas_call(
        _prep_body,
        grid=(_TC_GRID,),
        in_specs=[
            _qspec,
            pl.BlockSpec((_BN, EMB_DIM), lambda i: (i, 0)),
        ],
        out_specs=[
            pl.BlockSpec((_BN, 1), lambda i: (i, 0)),
            pl.BlockSpec((_BN, 1), lambda i: (i, 0)),
            _qspec,
            _qspec,
        ],
        out_shape=[
            jax.ShapeDtypeStruct((NODE_PAD, 1), _f32),
            jax.ShapeDtypeStruct((NODE_PAD, 1), _f32),
            _q_shape,
            _q_shape,
        ],
    )(deg, emb_p)


def _scale_body(ta_ref, tb_ref, dsq_ref, ua_ref, ub_ref):
    dsq = dsq_ref[...][None]
    ua_ref[...] = dsq * ta_ref[...]
    ub_ref[...] = dsq * tb_ref[...]


def _scale_call(ta, tb, dsq2):
    return pl.pallas_call(
        _scale_body,
        grid=(_TC_GRID,),
        in_specs=[_qspec, _qspec, pl.BlockSpec((_BN, 1), lambda i: (i, 0))],
        out_specs=[_qspec, _qspec],
        out_shape=[_q_shape, _q_shape],
    )(ta, tb, dsq2)


def _final_body(emb_ref, d_ref, t1a, t1b, t2a, t2b, t3a, t3b, out_ref):
    sa = t1a[...] + t2a[...] + t3a[...]
    sb = t1b[...] + t2b[...] + t3b[...]
    cat = jnp.concatenate([sa[0], sa[1], sb[0], sb[1]], axis=-1)
    out_ref[...] = 0.25 * (emb_ref[...] + d_ref[...] * cat)


def _final_call(emb_p, d2, ts):
    return pl.pallas_call(
        _final_body,
        grid=(_TC_GRID,),
        in_specs=[
            pl.BlockSpec((_BN, EMB_DIM), lambda i: (i, 0)),
            pl.BlockSpec((_BN, 1), lambda i: (i, 0)),
        ]
        + [_qspec] * 6,
        out_specs=pl.BlockSpec((_BN, EMB_DIM), lambda i: (i, 0)),
        out_shape=jax.ShapeDtypeStruct((NODE_PAD, EMB_DIM), _f32),
    )(emb_p, d2, *ts)


@jax.jit
def kernel(edge_index, emb_weight):
    row = edge_index[0]
    col = edge_index[1]
    pad = EDGE_PAD - N_EDGES
    row2d = jnp.concatenate([row, jnp.zeros((pad,), jnp.int32)]).reshape(BLOCKS, 128)
    col2d = jnp.concatenate([col, jnp.full((pad,), DUMMY, jnp.int32)]).reshape(
        BLOCKS, 128
    )
    emb_p = jnp.zeros((NODE_PAD, EMB_DIM), _f32).at[:N_NODES].set(emb_weight)

    ones_u = jnp.ones((NC, NODE_PAD, QW), _f32)
    deg = _spmm_kernel(ones_u, col2d, col2d)
    d2, dsq2, ua, ub = _prep_call(deg, emb_p)

    ts = []
    for layer in range(N_LAYERS):
        ta = _spmm_kernel(ua, row2d, col2d)
        tb = _spmm_kernel(ub, row2d, col2d)
        ts += [ta, tb]
        if layer < N_LAYERS - 1:
            ua, ub = _scale_call(ta, tb, dsq2)

    out = _final_call(emb_p, d2, ts)
    return out[:N_NODES]
